# baseline (speedup 1.0000x reference)
import jax
import jax.numpy as jnp
from jax import lax
from jax.experimental import pallas as pl
from jax.experimental.pallas import tpu as pltpu


def _fw_body(a_ref, o_ref):
    n = a_ref.shape[-1]
    a = a_ref[0]
    rows = lax.broadcasted_iota(jnp.int32, (n, n), 0)
    cols = lax.broadcasted_iota(jnp.int32, (n, n), 1)
    eye = rows == cols
    w = jnp.where((a != 0.0) | eye, a, jnp.inf)
    d0 = jnp.where(eye, 0.0, w)

    o_ref[0] = d0

    B = 8

    C = 64

    def block(kb, _):
        base = kb * B
        p = pltpu.roll(o_ref[0, pl.ds(base, B), :], -base, axis=1)
        for t in range(B):
            p = jnp.minimum(p, p[:, t : t + 1] + p[t : t + 1, :])
        r = pltpu.roll(p, base, axis=1)
        for s in range(n // C):
            d = o_ref[0, pl.ds(s * C, C), :]
            c0 = pltpu.roll(d, -base, axis=1)[:, 0:B]
            for t in range(B):
                d = jnp.minimum(d, c0[:, t : t + 1] + r[t : t + 1, :])
            o_ref[0, pl.ds(s * C, C), :] = d
        return 0

    lax.fori_loop(0, n // B, block, 0)


def kernel(adj):
    n = adj.shape[-1]
    batch = adj.shape[0] * adj.shape[1]
    a = adj.reshape(batch, n, n)
    out = pl.pallas_call(
        _fw_body,
        grid=(batch,),
        in_specs=[pl.BlockSpec((1, n, n), lambda b: (b, 0, 0))],
        out_specs=pl.BlockSpec((1, n, n), lambda b: (b, 0, 0)),
        out_shape=jax.ShapeDtypeStruct((batch, n, n), adj.dtype),
    )(a)
    return out.reshape(adj.shape)
